# baseline (speedup 1.0000x reference)
import functools

import jax
import jax.numpy as jnp
from jax import lax
from jax.experimental import pallas as pl
from jax.experimental.pallas import tpu as pltpu
from jax.experimental.pallas import tpu_sc as plsc

B = 16384
D = 64
NN = 8
NSMALL = 18

NC = 2
NS = 16
NW = NC * NS
BPW = B // NW

_sc_mesh = plsc.VectorSubcoreMesh(core_axis_name="c", subcore_axis_name="s")


@functools.partial(
    pl.kernel,
    out_type=jax.ShapeDtypeStruct((B, D), jnp.float32),
    mesh=_sc_mesh,
    scratch_types=[
        pltpu.VMEM((BPW,), jnp.int32),
        pltpu.VMEM((BPW, D), jnp.float32),
        pltpu.SemaphoreType.DMA,
    ],
)
def _station_gather(idx_hbm, table_hbm, out_hbm, idx_v, rows_v, sem):
    wid = lax.axis_index("s") * NC + lax.axis_index("c")
    base = wid * BPW
    pltpu.sync_copy(idx_hbm.at[pl.ds(base, BPW)], idx_v)

    def issue(i, carry):
        vec = idx_v[pl.ds(i * 16, 16)]
        for k in range(16):
            pltpu.make_async_copy(
                table_hbm.at[vec[k]], rows_v.at[i * 16 + k], sem).start()
        return carry

    lax.fori_loop(0, BPW // 16, issue, 0)
    pltpu.make_async_copy(table_hbm.at[pl.ds(0, BPW)], rows_v, sem).wait()
    pltpu.sync_copy(rows_v, out_hbm.at[pl.ds(base, BPW)])


_SQRT_HALF = 0.7071067811865476


def _gelu(x):
    return x * (0.5 * (1.0 + lax.erf(x * _SQRT_HALF)))


BB = 2048


def _tc_partial(num_ref, idx_ref, wnum_ref, bnum_ref,
                es_ref, et_ref, ew_ref, wcomb_ref, bcomb_ref, out_ref):
    x = num_ref[...]
    h = _gelu(jnp.dot(x, wnum_ref[...], preferred_element_type=jnp.float32)
              + bnum_ref[...])
    idx = idx_ref[...]
    iota = lax.broadcasted_iota(jnp.int32, (BB, NSMALL), 1)
    oh = ((idx[:, 0:1] == iota).astype(jnp.float32)
          + (idx[:, 1:2] == iota).astype(jnp.float32)
          + (idx[:, 2:3] == iota).astype(jnp.float32))
    small = jnp.concatenate([es_ref[...], et_ref[...], ew_ref[...]], axis=0)
    cat = jnp.dot(oh, small, preferred_element_type=jnp.float32) * 0.25
    wc = wcomb_ref[...]
    out_ref[...] = (jnp.dot(h, wc[:D], preferred_element_type=jnp.float32)
                    + jnp.dot(cat, wc[D:], preferred_element_type=jnp.float32)
                    + bcomb_ref[...])


def _tc_combine(part_ref, est_ref, wcomb_ref, out_ref):
    est = est_ref[...] * 0.25
    y = part_ref[...] + jnp.dot(est, wcomb_ref[D:],
                                preferred_element_type=jnp.float32)
    out_ref[...] = _gelu(y)


def kernel(numerical, season, time_period, weather_condition, station_id,
           W_num, b_num, emb_season, emb_time, emb_weather, emb_station,
           W_comb, b_comb):
    e_station = _station_gather(station_id, emb_station)
    idx_small = jnp.stack(
        [season, time_period + 4, weather_condition + 8], axis=1)
    part = pl.pallas_call(
        _tc_partial,
        grid=(B // BB,),
        in_specs=[
            pl.BlockSpec((BB, NN), lambda i: (i, 0)),
            pl.BlockSpec((BB, 3), lambda i: (i, 0)),
            pl.BlockSpec((NN, D), lambda i: (0, 0)),
            pl.BlockSpec((1, D), lambda i: (0, 0)),
            pl.BlockSpec((4, D), lambda i: (0, 0)),
            pl.BlockSpec((4, D), lambda i: (0, 0)),
            pl.BlockSpec((10, D), lambda i: (0, 0)),
            pl.BlockSpec((2 * D, D), lambda i: (0, 0)),
            pl.BlockSpec((1, D), lambda i: (0, 0)),
        ],
        out_specs=pl.BlockSpec((BB, D), lambda i: (i, 0)),
        out_shape=jax.ShapeDtypeStruct((B, D), jnp.float32),
    )(numerical, idx_small, W_num, b_num.reshape(1, D),
      emb_season, emb_time, emb_weather, W_comb, b_comb.reshape(1, D))
    out = pl.pallas_call(
        _tc_combine,
        grid=(B // BB,),
        in_specs=[
            pl.BlockSpec((BB, D), lambda i: (i, 0)),
            pl.BlockSpec((BB, D), lambda i: (i, 0)),
            pl.BlockSpec((2 * D, D), lambda i: (0, 0)),
        ],
        out_specs=pl.BlockSpec((BB, D), lambda i: (i, 0)),
        out_shape=jax.ShapeDtypeStruct((B, D), jnp.float32),
    )(part, e_station, W_comb)
    return out
